# baseline (speedup 1.0000x reference)
import jax
import jax.numpy as jnp
from jax import lax
from jax.experimental import pallas as pl
from jax.experimental.pallas import tpu as pltpu
from jax.experimental.pallas import tpu_sc as plsc

B, L, D = 4096, 200, 64
N = B * L
NC, NS = 2, 16
NW = NC * NS
ROWS_PER_W = N // NW
GATHER = 128
N_ITERS = ROWS_PER_W // GATHER
IDX_ROWS = ROWS_PER_W // GATHER
N_PAIR = 1_000_000 // 2


def _gather_body(idx_hbm, table_hbm, out_hbm, idx_v, sidx, rows_v,
                 g_sem0, g_sem1, s_sem0, s_sem1):
    wid = lax.axis_index("s") * NC + lax.axis_index("c")
    out_base = wid * ROWS_PER_W
    g_sems = (g_sem0, g_sem1)
    s_sems = (s_sem0, s_sem1)

    def make_sidx(t, buf):
        for k in range(8):
            v = idx_v[t, pl.ds(16 * k, 16)]
            sidx[buf, pl.ds(16 * k, 16)] = lax.shift_right_logical(v, 1)

    def issue_gather(t, buf):
        make_sidx(t, buf)
        pltpu.async_copy(table_hbm.at[sidx.at[buf]],
                         rows_v.at[buf], g_sems[buf])

    def wait_gather(buf):
        pltpu.make_async_copy(table_hbm.at[sidx.at[buf]],
                              rows_v.at[buf], g_sems[buf]).wait()

    def issue_store(t, buf):
        pltpu.async_copy(rows_v.at[buf],
                         out_hbm.at[pl.ds(out_base + t * GATHER, GATHER)],
                         s_sems[buf])

    def wait_store(buf):
        pltpu.make_async_copy(rows_v.at[buf],
                              out_hbm.at[pl.ds(out_base, GATHER)],
                              s_sems[buf]).wait()

    pltpu.sync_copy(idx_hbm.at[pl.ds(wid * IDX_ROWS, IDX_ROWS)], idx_v)

    issue_gather(0, 0)
    issue_gather(1, 1)

    def body(tt, carry):
        t0 = tt * 2
        t1 = t0 + 1
        wait_gather(0)
        issue_store(t0 - 2, 0)
        wait_gather(1)
        issue_store(t1 - 2, 1)
        wait_store(0)
        issue_gather(t0, 0)
        wait_store(1)
        issue_gather(t1, 1)
        return carry

    lax.fori_loop(1, N_ITERS // 2, body, 0)

    wait_gather(0)
    issue_store(N_ITERS - 2, 0)
    wait_gather(1)
    issue_store(N_ITERS - 1, 1)
    wait_store(0)
    wait_store(1)


def _transpose_body(g_ref, i_ref, o_ref):
    blk = g_ref[...]
    row = i_ref[0, pl.program_id(1)]
    odd = (row & 1) == 1
    t0 = blk[:, :64].T
    t1 = blk[:, 64:].T
    o_ref[0, :, 0] = jnp.where(odd[None, :], t1, t0).reshape(8, 8, 128)


def kernel(x, W):
    idx = x.T.reshape(N // GATHER, GATHER).astype(jnp.int32)
    table = W.reshape(N_PAIR, 128)
    mesh = plsc.VectorSubcoreMesh(core_axis_name="c", subcore_axis_name="s")
    run = pl.kernel(
        _gather_body,
        out_type=jax.ShapeDtypeStruct((N, 128), jnp.float32),
        mesh=mesh,
        scratch_types=[
            pltpu.VMEM((IDX_ROWS, GATHER), jnp.int32),
            pltpu.VMEM((2, GATHER), jnp.int32),
            pltpu.VMEM((2, GATHER, 128), jnp.float32),
            pltpu.SemaphoreType.DMA,
            pltpu.SemaphoreType.DMA,
            pltpu.SemaphoreType.DMA,
            pltpu.SemaphoreType.DMA,
        ],
        compiler_params=pltpu.CompilerParams(use_tc_tiling_on_sc=True),
    )
    g = run(idx, table)

    out5 = pl.pallas_call(
        _transpose_body,
        grid=(L, B // 128),
        in_specs=[
            pl.BlockSpec((128, 128), lambda l, b: (l * 32 + b, 0)),
            pl.BlockSpec((1, 32, 128), lambda l, b: (l, 0, 0)),
        ],
        out_specs=pl.BlockSpec((1, 8, 1, 8, 128), lambda l, b: (l, 0, b, 0, 0)),
        out_shape=jax.ShapeDtypeStruct((L, 8, B // 128, 8, 128), jnp.float32),
    )(g, idx.reshape(L, B // 128, 128))

    return out5.transpose(2, 4, 0, 1, 3).reshape(B, L, D)
